# baseline (speedup 1.0000x reference)
import functools

import jax
import jax.numpy as jnp
from jax import lax
from jax.experimental import pallas as pl
from jax.experimental.pallas import tpu as pltpu
from jax.experimental.pallas import tpu_sc as plsc

N_NODES_ = 10000
N_PAD = 10240
E_EDGES = 320000
E_PAD = 327680
NC, NS = 2, 16
NW = NC * NS
EPT = E_PAD // NW
ROWS_PT = EPT // 128
STRIPE = N_PAD // NS

_sc_mesh = functools.partial(
    plsc.VectorSubcoreMesh, core_axis_name="c", subcore_axis_name="s")


@functools.partial(
    pl.kernel,
    out_type=jax.ShapeDtypeStruct((NC, N_PAD, 16), jnp.float32),
    mesh=_sc_mesh(),
    compiler_params=pltpu.CompilerParams(use_tc_tiling_on_sc=False),
    scratch_types=[
        pltpu.VMEM((ROWS_PT, 128), jnp.int32),
        pltpu.VMEM((128, 16), jnp.float32),
        pltpu.VMEM((128, 16), jnp.float32),
        pltpu.VMEM_SHARED((N_PAD, 16), jnp.float32),
    ],
)
def _deg_kernel(dst_hbm, out_hbm, didx, ones_v, zeros_v, acc):
    c = lax.axis_index("c")
    s = lax.axis_index("s")
    wid = s * NC + c

    def fill_body(i, _):
        ones_v[i, :] = jnp.ones((16,), jnp.float32)
        zeros_v[i, :] = jnp.zeros((16,), jnp.float32)
        return _

    lax.fori_loop(0, 128, fill_body, None)

    def zcopy_body(r, _):
        pltpu.sync_copy(zeros_v, acc.at[pl.ds(s * STRIPE + r * 128, 128)])
        return _

    lax.fori_loop(0, STRIPE // 128, zcopy_body, None)
    plsc.subcore_barrier()
    pltpu.sync_copy(dst_hbm.at[pl.ds(wid * ROWS_PT, ROWS_PT)], didx)

    def body(b, _):
        pltpu.sync_copy(ones_v, acc.at[didx.at[b]], add=True)
        return _

    lax.fori_loop(0, ROWS_PT, body, None)
    plsc.subcore_barrier()
    pltpu.sync_copy(acc.at[pl.ds(s * STRIPE, STRIPE)],
                    out_hbm.at[c, pl.ds(s * STRIPE, STRIPE)])


_NBUF = 2
_F = 64
_STEADY = ROWS_PT - _NBUF


@functools.partial(
    pl.kernel,
    out_type=jax.ShapeDtypeStruct((NC, N_PAD, _F), jnp.float32),
    mesh=_sc_mesh(),
    compiler_params=pltpu.CompilerParams(use_tc_tiling_on_sc=False),
    scratch_types=(
        [pltpu.VMEM((ROWS_PT, 128), jnp.int32)] * 2
        + [pltpu.VMEM((128, _F), jnp.float32)] * _NBUF
        + [pltpu.VMEM_SHARED((N_PAD, _F), jnp.float32)]
        + [pltpu.VMEM_SHARED((N_PAD, _F), jnp.float32)]
        + [pltpu.SemaphoreType.DMA] * (2 * _NBUF + 2)
    ),
)
def _agg64(src_hbm, dst_hbm, p_hbm, out_hbm, sidx, didx, *scr):
    rows = scr[:_NBUF]
    p_sp = scr[_NBUF]
    acc = scr[_NBUF + 1]
    gsem = scr[_NBUF + 2:2 * _NBUF + 2]
    ssem = scr[2 * _NBUF + 2:3 * _NBUF + 2]
    isem = scr[3 * _NBUF + 2]
    psem = scr[3 * _NBUF + 3]
    c = lax.axis_index("c")
    s = lax.axis_index("s")
    wid = s * NC + c

    pltpu.async_copy(src_hbm.at[pl.ds(wid * ROWS_PT, ROWS_PT)], sidx, isem)
    pltpu.async_copy(dst_hbm.at[pl.ds(wid * ROWS_PT, ROWS_PT)], didx, isem)
    pltpu.async_copy(p_hbm.at[pl.ds(s * STRIPE, STRIPE)],
                     p_sp.at[pl.ds(s * STRIPE, STRIPE)], psem)

    nvec = _F // 16

    def zero_body(i, _):
        rows[0][i // nvec, pl.ds((i % nvec) * 16, 16)] = jnp.zeros(
            (16,), jnp.float32)
        return _

    lax.fori_loop(0, 128 * nvec, zero_body, None)

    def zcopy_body(r, _):
        pltpu.sync_copy(rows[0], acc.at[pl.ds(s * STRIPE + r * 128, 128)])
        return _

    lax.fori_loop(0, STRIPE // 128, zcopy_body, None)
    pltpu.make_async_copy(
        src_hbm.at[pl.ds(wid * ROWS_PT, ROWS_PT)], sidx, isem).wait()
    pltpu.make_async_copy(
        dst_hbm.at[pl.ds(wid * ROWS_PT, ROWS_PT)], didx, isem).wait()
    pltpu.make_async_copy(
        p_hbm.at[pl.ds(s * STRIPE, STRIPE)],
        p_sp.at[pl.ds(s * STRIPE, STRIPE)], psem).wait()
    plsc.subcore_barrier()
    for j in range(_NBUF):
        pltpu.async_copy(p_sp.at[sidx.at[j]], rows[j], gsem[j])

    def steady(g, _):
        for j in range(_NBUF):
            b = g * _NBUF + j
            pltpu.make_async_copy(
                p_sp.at[sidx.at[b]], rows[j], gsem[j]).wait()
            pltpu.async_copy(rows[j], acc.at[didx.at[b]], ssem[j], add=True)
        for j in range(_NBUF):
            b = g * _NBUF + j
            pltpu.make_async_copy(rows[j], acc.at[didx.at[b]], ssem[j]).wait()
            pltpu.async_copy(p_sp.at[sidx.at[b + _NBUF]], rows[j], gsem[j])
        return _

    lax.fori_loop(0, _STEADY // _NBUF, steady, None)
    for j in range(_NBUF):
        b = _STEADY + j
        pltpu.make_async_copy(p_sp.at[sidx.at[b]], rows[j], gsem[j]).wait()
        pltpu.async_copy(rows[j], acc.at[didx.at[b]], ssem[j], add=True)
    for j in range(_NBUF):
        b = _STEADY + j
        pltpu.make_async_copy(rows[j], acc.at[didx.at[b]], ssem[j]).wait()
    plsc.subcore_barrier()
    pltpu.sync_copy(acc.at[pl.ds(s * STRIPE, STRIPE)],
                    out_hbm.at[c, pl.ds(s * STRIPE, STRIPE)])


_BLK = 1024


def _s_from_deg(deg_ref):
    d = deg_ref[0] + deg_ref[1]
    return lax.rsqrt(
        1.0 + jnp.sum(d, axis=1, keepdims=True) * (1.0 / 16.0))


def _scale_body(degT_ref, x_ref, pa_ref, pb_ref):
    s = _s_from_deg(degT_ref)
    p = s * x_ref[...]
    pa_ref[...] = p[:, :64]
    pb_ref[...] = p[:, 64:]


def _mid_body(agga_ref, aggb_ref, degT_ref, x_ref, w1a_ref, w1b_ref, b1_ref,
              w2_ref, z_ref, p2_ref):
    s = _s_from_deg(degT_ref)
    s2 = s * s
    x = x_ref[...]
    qa = s * (agga_ref[0] + agga_ref[1]) + s2 * x[:, :64]
    qb = s * (aggb_ref[0] + aggb_ref[1]) + s2 * x[:, 64:]
    h = jnp.dot(qa, w1a_ref[...], preferred_element_type=jnp.float32)
    h += jnp.dot(qb, w1b_ref[...], preferred_element_type=jnp.float32)
    h = jnp.maximum(h + b1_ref[...], 0.0)
    z = jnp.dot(h, w2_ref[...], preferred_element_type=jnp.float32)
    z_ref[...] = z
    p2_ref[...] = s * z


def _final_body(agg_ref, degT_ref, z_ref, b2_ref, out_ref):
    s = _s_from_deg(degT_ref)
    out_ref[...] = s * (agg_ref[0] + agg_ref[1]) + (s * s) * z_ref[...] \
        + b2_ref[...]


def _row_spec(f, blk=_BLK):
    return pl.BlockSpec((blk, f), lambda i: (i, 0))


def _agg_spec(f, blk=_BLK):
    return pl.BlockSpec((NC, blk, f), lambda i: (0, i, 0))


def _full_spec(a, b):
    return pl.BlockSpec((a, b), lambda i: (0, 0))


_GRID = (N_PAD // _BLK,)


def _tc_scale(deg_parts, x_pad):
    return pl.pallas_call(
        _scale_body,
        grid=_GRID,
        in_specs=[_agg_spec(16), _row_spec(128)],
        out_specs=[_row_spec(64), _row_spec(64)],
        out_shape=[jax.ShapeDtypeStruct((N_PAD, 64), jnp.float32),
                   jax.ShapeDtypeStruct((N_PAD, 64), jnp.float32)],
    )(deg_parts, x_pad)


def _tc_mid(agg1a, agg1b, deg_parts, x_pad, W1a, W1b, b1, W2):
    return pl.pallas_call(
        _mid_body,
        grid=_GRID,
        in_specs=[_agg_spec(64), _agg_spec(64), _agg_spec(16), _row_spec(128),
                  _full_spec(64, 256), _full_spec(64, 256), _full_spec(1, 256),
                  _full_spec(256, 64)],
        out_specs=[_row_spec(64), _row_spec(64)],
        out_shape=[jax.ShapeDtypeStruct((N_PAD, 64), jnp.float32),
                   jax.ShapeDtypeStruct((N_PAD, 64), jnp.float32)],
    )(agg1a, agg1b, deg_parts, x_pad, W1a, W1b, b1, W2)


_FBLK = 1000


def _tc_final(agg2, deg_parts, z, b2):
    return pl.pallas_call(
        _final_body,
        grid=(N_NODES_ // _FBLK,),
        in_specs=[_agg_spec(64, _FBLK), _agg_spec(16, _FBLK),
                  _row_spec(64, _FBLK), _full_spec(1, 64)],
        out_specs=_row_spec(64, _FBLK),
        out_shape=jax.ShapeDtypeStruct((N_NODES_, 64), jnp.float32),
    )(agg2, deg_parts, z, b2)


def kernel(x, edge_index, W1, b1, W2, b2):
    ei = edge_index.astype(jnp.int32)
    pad = jnp.full((E_PAD - E_EDGES,), N_NODES_, dtype=jnp.int32)
    src2d = jnp.concatenate([ei[0], pad]).reshape(E_PAD // 128, 128)
    dst2d = jnp.concatenate([ei[1], pad]).reshape(E_PAD // 128, 128)
    x_pad = jnp.pad(x, ((0, N_PAD - N_NODES_), (0, 0)))

    deg_parts = _deg_kernel(dst2d)

    p1a, p1b = _tc_scale(deg_parts, x_pad)
    agg1a = _agg64(src2d, dst2d, p1a)
    agg1b = _agg64(src2d, dst2d, p1b)
    z, p2 = _tc_mid(agg1a, agg1b, deg_parts, x_pad, W1[:64], W1[64:],
                    b1.reshape(1, 256), W2)
    agg2 = _agg64(src2d, dst2d, p2)
    return _tc_final(agg2, deg_parts, z, b2.reshape(1, 64))


# speedup vs baseline: 1.0523x; 1.0523x over previous
import functools

import jax
import jax.numpy as jnp
from jax import lax
from jax.experimental import pallas as pl
from jax.experimental.pallas import tpu as pltpu
from jax.experimental.pallas import tpu_sc as plsc

N_NODES_ = 10000
N_PAD = 10240
E_EDGES = 320000
E_PAD = 327680
NC, NS = 2, 16
NW = NC * NS
EPT = E_PAD // NW
ROWS_PT = EPT // 128
STRIPE = N_PAD // NS

_sc_mesh = functools.partial(
    plsc.VectorSubcoreMesh, core_axis_name="c", subcore_axis_name="s")


@functools.partial(
    pl.kernel,
    out_type=jax.ShapeDtypeStruct((NC, N_PAD, 16), jnp.float32),
    mesh=_sc_mesh(),
    compiler_params=pltpu.CompilerParams(use_tc_tiling_on_sc=False),
    scratch_types=[
        pltpu.VMEM((ROWS_PT, 128), jnp.int32),
        pltpu.VMEM((128, 16), jnp.float32),
        pltpu.VMEM((128, 16), jnp.float32),
        pltpu.VMEM_SHARED((N_PAD, 16), jnp.float32),
    ],
)
def _deg_kernel(dst_hbm, out_hbm, didx, ones_v, zeros_v, acc):
    c = lax.axis_index("c")
    s = lax.axis_index("s")
    wid = s * NC + c

    def fill_body(i, _):
        ones_v[i, :] = jnp.ones((16,), jnp.float32)
        zeros_v[i, :] = jnp.zeros((16,), jnp.float32)
        return _

    lax.fori_loop(0, 128, fill_body, None)

    def zcopy_body(r, _):
        pltpu.sync_copy(zeros_v, acc.at[pl.ds(s * STRIPE + r * 128, 128)])
        return _

    lax.fori_loop(0, STRIPE // 128, zcopy_body, None)
    plsc.subcore_barrier()
    pltpu.sync_copy(dst_hbm.at[pl.ds(wid * ROWS_PT, ROWS_PT)], didx)

    def body(b, _):
        pltpu.sync_copy(ones_v, acc.at[didx.at[b]], add=True)
        return _

    lax.fori_loop(0, ROWS_PT, body, None)
    plsc.subcore_barrier()
    pltpu.sync_copy(acc.at[pl.ds(s * STRIPE, STRIPE)],
                    out_hbm.at[c, pl.ds(s * STRIPE, STRIPE)])


_NBUF = 2
_F = 64
_STEADY = ROWS_PT - _NBUF


@functools.partial(
    pl.kernel,
    out_type=jax.ShapeDtypeStruct((NC, N_PAD, _F), jnp.float32),
    mesh=_sc_mesh(),
    compiler_params=pltpu.CompilerParams(use_tc_tiling_on_sc=False),
    scratch_types=(
        [pltpu.VMEM((ROWS_PT, 128), jnp.int32)] * 2
        + [pltpu.VMEM((128, _F), jnp.float32)] * _NBUF
        + [pltpu.VMEM_SHARED((N_PAD, _F), jnp.float32)]
        + [pltpu.VMEM_SHARED((N_PAD, _F), jnp.float32)]
        + [pltpu.SemaphoreType.DMA] * (2 * _NBUF + 2)
    ),
)
def _agg64(src_hbm, dst_hbm, p_hbm, out_hbm, sidx, didx, *scr):
    rows = scr[:_NBUF]
    p_sp = scr[_NBUF]
    acc = scr[_NBUF + 1]
    gsem = scr[_NBUF + 2:2 * _NBUF + 2]
    ssem = scr[2 * _NBUF + 2:3 * _NBUF + 2]
    isem = scr[3 * _NBUF + 2]
    psem = scr[3 * _NBUF + 3]
    c = lax.axis_index("c")
    s = lax.axis_index("s")
    wid = s * NC + c

    pltpu.async_copy(src_hbm.at[pl.ds(wid * ROWS_PT, ROWS_PT)], sidx, isem)
    pltpu.async_copy(dst_hbm.at[pl.ds(wid * ROWS_PT, ROWS_PT)], didx, isem)
    pltpu.async_copy(p_hbm.at[pl.ds(s * STRIPE, STRIPE)],
                     p_sp.at[pl.ds(s * STRIPE, STRIPE)], psem)

    nvec = _F // 16

    def zero_body(i, _):
        rows[0][i // nvec, pl.ds((i % nvec) * 16, 16)] = jnp.zeros(
            (16,), jnp.float32)
        return _

    lax.fori_loop(0, 128 * nvec, zero_body, None)

    def zcopy_body(r, _):
        pltpu.sync_copy(rows[0], acc.at[pl.ds(s * STRIPE + r * 128, 128)])
        return _

    lax.fori_loop(0, STRIPE // 128, zcopy_body, None)
    pltpu.make_async_copy(
        src_hbm.at[pl.ds(wid * ROWS_PT, ROWS_PT)], sidx, isem).wait()
    pltpu.make_async_copy(
        dst_hbm.at[pl.ds(wid * ROWS_PT, ROWS_PT)], didx, isem).wait()
    pltpu.make_async_copy(
        p_hbm.at[pl.ds(s * STRIPE, STRIPE)],
        p_sp.at[pl.ds(s * STRIPE, STRIPE)], psem).wait()
    plsc.subcore_barrier()
    for j in range(_NBUF):
        pltpu.async_copy(p_sp.at[sidx.at[j]], rows[j], gsem[j])

    def steady(g, _):
        for j in range(_NBUF):
            b = g * _NBUF + j
            pltpu.make_async_copy(
                p_sp.at[sidx.at[b]], rows[j], gsem[j]).wait()
            pltpu.sync_copy(rows[j], acc.at[didx.at[b]], add=True)
            pltpu.async_copy(p_sp.at[sidx.at[b + _NBUF]], rows[j], gsem[j])
        return _

    lax.fori_loop(0, _STEADY // _NBUF, steady, None)
    for j in range(_NBUF):
        b = _STEADY + j
        pltpu.make_async_copy(p_sp.at[sidx.at[b]], rows[j], gsem[j]).wait()
        pltpu.sync_copy(rows[j], acc.at[didx.at[b]], add=True)
    plsc.subcore_barrier()
    pltpu.sync_copy(acc.at[pl.ds(s * STRIPE, STRIPE)],
                    out_hbm.at[c, pl.ds(s * STRIPE, STRIPE)])


_BLK = 1024


def _s_from_deg(deg_ref):
    d = deg_ref[0] + deg_ref[1]
    return lax.rsqrt(
        1.0 + jnp.sum(d, axis=1, keepdims=True) * (1.0 / 16.0))


def _scale_body(degT_ref, x_ref, pa_ref, pb_ref):
    s = _s_from_deg(degT_ref)
    p = s * x_ref[...]
    pa_ref[...] = p[:, :64]
    pb_ref[...] = p[:, 64:]


def _mid_body(agga_ref, aggb_ref, degT_ref, x_ref, w1a_ref, w1b_ref, b1_ref,
              w2_ref, z_ref, p2_ref):
    s = _s_from_deg(degT_ref)
    s2 = s * s
    x = x_ref[...]
    qa = s * (agga_ref[0] + agga_ref[1]) + s2 * x[:, :64]
    qb = s * (aggb_ref[0] + aggb_ref[1]) + s2 * x[:, 64:]
    h = jnp.dot(qa, w1a_ref[...], preferred_element_type=jnp.float32)
    h += jnp.dot(qb, w1b_ref[...], preferred_element_type=jnp.float32)
    h = jnp.maximum(h + b1_ref[...], 0.0)
    z = jnp.dot(h, w2_ref[...], preferred_element_type=jnp.float32)
    z_ref[...] = z
    p2_ref[...] = s * z


def _final_body(agg_ref, degT_ref, z_ref, b2_ref, out_ref):
    s = _s_from_deg(degT_ref)
    out_ref[...] = s * (agg_ref[0] + agg_ref[1]) + (s * s) * z_ref[...] \
        + b2_ref[...]


def _row_spec(f, blk=_BLK):
    return pl.BlockSpec((blk, f), lambda i: (i, 0))


def _agg_spec(f, blk=_BLK):
    return pl.BlockSpec((NC, blk, f), lambda i: (0, i, 0))


def _full_spec(a, b):
    return pl.BlockSpec((a, b), lambda i: (0, 0))


_GRID = (N_PAD // _BLK,)


def _tc_scale(deg_parts, x_pad):
    return pl.pallas_call(
        _scale_body,
        grid=_GRID,
        in_specs=[_agg_spec(16), _row_spec(128)],
        out_specs=[_row_spec(64), _row_spec(64)],
        out_shape=[jax.ShapeDtypeStruct((N_PAD, 64), jnp.float32),
                   jax.ShapeDtypeStruct((N_PAD, 64), jnp.float32)],
    )(deg_parts, x_pad)


def _tc_mid(agg1a, agg1b, deg_parts, x_pad, W1a, W1b, b1, W2):
    return pl.pallas_call(
        _mid_body,
        grid=_GRID,
        in_specs=[_agg_spec(64), _agg_spec(64), _agg_spec(16), _row_spec(128),
                  _full_spec(64, 256), _full_spec(64, 256), _full_spec(1, 256),
                  _full_spec(256, 64)],
        out_specs=[_row_spec(64), _row_spec(64)],
        out_shape=[jax.ShapeDtypeStruct((N_PAD, 64), jnp.float32),
                   jax.ShapeDtypeStruct((N_PAD, 64), jnp.float32)],
    )(agg1a, agg1b, deg_parts, x_pad, W1a, W1b, b1, W2)


_FBLK = 1000


def _tc_final(agg2, deg_parts, z, b2):
    return pl.pallas_call(
        _final_body,
        grid=(N_NODES_ // _FBLK,),
        in_specs=[_agg_spec(64, _FBLK), _agg_spec(16, _FBLK),
                  _row_spec(64, _FBLK), _full_spec(1, 64)],
        out_specs=_row_spec(64, _FBLK),
        out_shape=jax.ShapeDtypeStruct((N_NODES_, 64), jnp.float32),
    )(agg2, deg_parts, z, b2)


def kernel(x, edge_index, W1, b1, W2, b2):
    ei = edge_index.astype(jnp.int32)
    pad = jnp.full((E_PAD - E_EDGES,), N_NODES_, dtype=jnp.int32)
    src2d = jnp.concatenate([ei[0], pad]).reshape(E_PAD // 128, 128)
    dst2d = jnp.concatenate([ei[1], pad]).reshape(E_PAD // 128, 128)
    x_pad = jnp.pad(x, ((0, N_PAD - N_NODES_), (0, 0)))

    deg_parts = _deg_kernel(dst2d)

    p1a, p1b = _tc_scale(deg_parts, x_pad)
    agg1a = _agg64(src2d, dst2d, p1a)
    agg1b = _agg64(src2d, dst2d, p1b)
    z, p2 = _tc_mid(agg1a, agg1b, deg_parts, x_pad, W1[:64], W1[64:],
                    b1.reshape(1, 256), W2)
    agg2 = _agg64(src2d, dst2d, p2)
    return _tc_final(agg2, deg_parts, z, b2.reshape(1, 64))


# speedup vs baseline: 1.0834x; 1.0296x over previous
import functools

import jax
import jax.numpy as jnp
from jax import lax
from jax.experimental import pallas as pl
from jax.experimental.pallas import tpu as pltpu
from jax.experimental.pallas import tpu_sc as plsc

N_NODES_ = 10000
N_PAD = 10240
E_EDGES = 320000
E_PAD = 327680
NC, NS = 2, 16
NW = NC * NS
EPT = E_PAD // NW
ROWS_PT = EPT // 128
STRIPE = N_PAD // NS

_sc_mesh = functools.partial(
    plsc.VectorSubcoreMesh, core_axis_name="c", subcore_axis_name="s")


_DW = 16


@functools.partial(
    pl.kernel,
    out_type=jax.ShapeDtypeStruct((NC, N_PAD, _DW), jnp.float32),
    mesh=_sc_mesh(),
    compiler_params=pltpu.CompilerParams(use_tc_tiling_on_sc=False),
    scratch_types=[
        pltpu.VMEM((ROWS_PT, 128), jnp.int32),
        pltpu.VMEM((128, _DW), jnp.float32),
        pltpu.VMEM((128, _DW), jnp.float32),
        pltpu.VMEM_SHARED((N_PAD, _DW), jnp.float32),
    ],
)
def _deg_kernel(dst_hbm, out_hbm, didx, ones_v, zeros_v, acc):
    c = lax.axis_index("c")
    s = lax.axis_index("s")
    wid = s * NC + c

    def fill_body(i, _):
        ones_v[i, :] = jnp.ones((16,), jnp.float32)
        zeros_v[i, :] = jnp.zeros((16,), jnp.float32)
        return _

    lax.fori_loop(0, 128, fill_body, None)

    def zcopy_body(r, _):
        pltpu.sync_copy(zeros_v, acc.at[pl.ds(s * STRIPE + r * 128, 128)])
        return _

    lax.fori_loop(0, STRIPE // 128, zcopy_body, None)
    plsc.subcore_barrier()
    pltpu.sync_copy(dst_hbm.at[pl.ds(wid * ROWS_PT, ROWS_PT)], didx)

    def body(b, _):
        pltpu.sync_copy(ones_v, acc.at[didx.at[b]], add=True)
        return _

    lax.fori_loop(0, ROWS_PT, body, None)
    plsc.subcore_barrier()
    pltpu.sync_copy(acc.at[pl.ds(s * STRIPE, STRIPE)],
                    out_hbm.at[c, pl.ds(s * STRIPE, STRIPE)])


_NBUF = 2
_F = 64
_STEADY = ROWS_PT - _NBUF


def _make_agg64(col_off):
    @functools.partial(
        pl.kernel,
        out_type=jax.ShapeDtypeStruct((NC, N_PAD, _F), jnp.float32),
        mesh=_sc_mesh(),
        compiler_params=pltpu.CompilerParams(use_tc_tiling_on_sc=False),
        scratch_types=(
            [pltpu.VMEM((ROWS_PT, 128), jnp.int32)] * 2
            + [pltpu.VMEM((128, _F), jnp.float32)] * _NBUF
            + [pltpu.VMEM_SHARED((N_PAD, _F), jnp.float32)]
            + [pltpu.VMEM_SHARED((N_PAD, _F), jnp.float32)]
            + [pltpu.SemaphoreType.DMA] * (2 * _NBUF + 2)
        ),
    )
    def _agg64(src_hbm, dst_hbm, p_hbm, out_hbm, sidx, didx, *scr):
        rows = scr[:_NBUF]
        p_sp = scr[_NBUF]
        acc = scr[_NBUF + 1]
        gsem = scr[_NBUF + 2:2 * _NBUF + 2]
        ssem = scr[2 * _NBUF + 2:3 * _NBUF + 2]
        isem = scr[3 * _NBUF + 2]
        psem = scr[3 * _NBUF + 3]
        c = lax.axis_index("c")
        s = lax.axis_index("s")
        wid = s * NC + c

        pltpu.async_copy(src_hbm.at[pl.ds(wid * ROWS_PT, ROWS_PT)], sidx,
                         isem)
        pltpu.async_copy(dst_hbm.at[pl.ds(wid * ROWS_PT, ROWS_PT)], didx,
                         isem)
        pltpu.async_copy(
            p_hbm.at[pl.ds(s * STRIPE, STRIPE), pl.ds(col_off, _F)],
            p_sp.at[pl.ds(s * STRIPE, STRIPE)], psem)

        nvec = _F // 16

        def zero_body(i, _):
            rows[0][i // nvec, pl.ds((i % nvec) * 16, 16)] = jnp.zeros(
                (16,), jnp.float32)
            return _

        lax.fori_loop(0, 128 * nvec, zero_body, None)

        def zcopy_body(r, _):
            pltpu.sync_copy(rows[0], acc.at[pl.ds(s * STRIPE + r * 128, 128)])
            return _

        lax.fori_loop(0, STRIPE // 128, zcopy_body, None)
        pltpu.make_async_copy(
            src_hbm.at[pl.ds(wid * ROWS_PT, ROWS_PT)], sidx, isem).wait()
        pltpu.make_async_copy(
            dst_hbm.at[pl.ds(wid * ROWS_PT, ROWS_PT)], didx, isem).wait()
        pltpu.make_async_copy(
            p_hbm.at[pl.ds(s * STRIPE, STRIPE), pl.ds(col_off, _F)],
            p_sp.at[pl.ds(s * STRIPE, STRIPE)], psem).wait()
        plsc.subcore_barrier()
        for j in range(_NBUF):
            pltpu.async_copy(p_sp.at[sidx.at[j]], rows[j], gsem[j])

        def steady(g, _):
            for j in range(_NBUF):
                b = g * _NBUF + j
                pltpu.make_async_copy(
                    p_sp.at[sidx.at[b]], rows[j], gsem[j]).wait()
                pltpu.sync_copy(rows[j], acc.at[didx.at[b]], add=True)
                pltpu.async_copy(p_sp.at[sidx.at[b + _NBUF]], rows[j],
                                 gsem[j])
            return _

        lax.fori_loop(0, _STEADY // _NBUF, steady, None)
        for j in range(_NBUF):
            b = _STEADY + j
            pltpu.make_async_copy(
                p_sp.at[sidx.at[b]], rows[j], gsem[j]).wait()
            pltpu.sync_copy(rows[j], acc.at[didx.at[b]], add=True)
        plsc.subcore_barrier()
        pltpu.sync_copy(acc.at[pl.ds(s * STRIPE, STRIPE)],
                        out_hbm.at[c, pl.ds(s * STRIPE, STRIPE)])

    return _agg64


_agg64_lo = _make_agg64(0)
_agg64_hi = _make_agg64(64)


_BLK = 1024


def _s_from_deg(deg_ref):
    d = deg_ref[0] + deg_ref[1]
    return lax.rsqrt(
        1.0 + jnp.sum(d, axis=1, keepdims=True) * (1.0 / 16.0))


def _scale_body(degT_ref, x_ref, p_ref):
    s = _s_from_deg(degT_ref)
    p_ref[...] = s * x_ref[...]


def _mid_body(agga_ref, aggb_ref, degT_ref, x_ref, w1a_ref, w1b_ref, b1_ref,
              w2_ref, zp2_ref):
    s = _s_from_deg(degT_ref)
    s2 = s * s
    x = x_ref[...]
    qa = s * (agga_ref[0] + agga_ref[1]) + s2 * x[:, :64]
    qb = s * (aggb_ref[0] + aggb_ref[1]) + s2 * x[:, 64:]
    h = jnp.dot(qa, w1a_ref[...], preferred_element_type=jnp.float32)
    h += jnp.dot(qb, w1b_ref[...], preferred_element_type=jnp.float32)
    h = jnp.maximum(h + b1_ref[...], 0.0)
    z = jnp.dot(h, w2_ref[...], preferred_element_type=jnp.float32)
    zp2_ref[...] = jnp.concatenate([z, s * z], axis=1)


def _final_body(agg_ref, degT_ref, zp2_ref, b2_ref, out_ref):
    s = _s_from_deg(degT_ref)
    z = zp2_ref[:, :64]
    out_ref[...] = s * (agg_ref[0] + agg_ref[1]) + (s * s) * z + b2_ref[...]


def _row_spec(f, blk=_BLK):
    return pl.BlockSpec((blk, f), lambda i: (i, 0))


def _agg_spec(f, blk=_BLK):
    return pl.BlockSpec((NC, blk, f), lambda i: (0, i, 0))


def _full_spec(a, b):
    return pl.BlockSpec((a, b), lambda i: (0, 0))


_GRID = (N_PAD // _BLK,)


def _tc_scale(deg_parts, x_pad):
    return pl.pallas_call(
        _scale_body,
        grid=_GRID,
        in_specs=[_agg_spec(16), _row_spec(128)],
        out_specs=_row_spec(128),
        out_shape=jax.ShapeDtypeStruct((N_PAD, 128), jnp.float32),
    )(deg_parts, x_pad)


def _tc_mid(agg1a, agg1b, deg_parts, x_pad, W1a, W1b, b1, W2):
    return pl.pallas_call(
        _mid_body,
        grid=_GRID,
        in_specs=[_agg_spec(64), _agg_spec(64), _agg_spec(16), _row_spec(128),
                  _full_spec(64, 256), _full_spec(64, 256), _full_spec(1, 256),
                  _full_spec(256, 64)],
        out_specs=_row_spec(128),
        out_shape=jax.ShapeDtypeStruct((N_PAD, 128), jnp.float32),
    )(agg1a, agg1b, deg_parts, x_pad, W1a, W1b, b1, W2)


_FBLK = 1000


def _tc_final(agg2, deg_parts, zp2, b2):
    return pl.pallas_call(
        _final_body,
        grid=(N_NODES_ // _FBLK,),
        in_specs=[_agg_spec(64, _FBLK), _agg_spec(16, _FBLK),
                  _row_spec(128, _FBLK), _full_spec(1, 64)],
        out_specs=_row_spec(64, _FBLK),
        out_shape=jax.ShapeDtypeStruct((N_NODES_, 64), jnp.float32),
    )(agg2, deg_parts, zp2, b2)


def kernel(x, edge_index, W1, b1, W2, b2):
    ei = edge_index.astype(jnp.int32)
    pad = jnp.full((E_PAD - E_EDGES,), N_NODES_, dtype=jnp.int32)
    src2d = jnp.concatenate([ei[0], pad]).reshape(E_PAD // 128, 128)
    dst2d = jnp.concatenate([ei[1], pad]).reshape(E_PAD // 128, 128)
    x_pad = jnp.pad(x, ((0, N_PAD - N_NODES_), (0, 0)))

    deg_parts = _deg_kernel(dst2d)

    p1 = _tc_scale(deg_parts, x_pad)
    agg1a = _agg64_lo(src2d, dst2d, p1)
    agg1b = _agg64_hi(src2d, dst2d, p1)
    zp2 = _tc_mid(agg1a, agg1b, deg_parts, x_pad, W1[:64], W1[64:],
                  b1.reshape(1, 256), W2)
    agg2 = _agg64_hi(src2d, dst2d, zp2)
    return _tc_final(agg2, deg_parts, zp2, b2.reshape(1, 64))


# speedup vs baseline: 1.1257x; 1.0390x over previous
import functools

import jax
import jax.numpy as jnp
from jax import lax
from jax.experimental import pallas as pl
from jax.experimental.pallas import tpu as pltpu
from jax.experimental.pallas import tpu_sc as plsc

N_NODES_ = 10000
N_PAD = 10240
E_EDGES = 320000
E_ROWS = E_EDGES // 128
NC, NS = 2, 16
NW = NC * NS
ROWS_PT = E_ROWS // NW
N_XTRA = E_ROWS - NW * ROWS_PT
STRIPE = N_PAD // NS


def _row_base(wid):
    return jnp.where(wid < N_XTRA, wid * (ROWS_PT + 1),
                     wid * ROWS_PT + N_XTRA)

_sc_mesh = functools.partial(
    plsc.VectorSubcoreMesh, core_axis_name="c", subcore_axis_name="s")


_DW = 16


@functools.partial(
    pl.kernel,
    out_type=jax.ShapeDtypeStruct((NC, N_PAD, _DW), jnp.float32),
    mesh=_sc_mesh(),
    compiler_params=pltpu.CompilerParams(use_tc_tiling_on_sc=False),
    scratch_types=[
        pltpu.VMEM((ROWS_PT + 1, 128), jnp.int32),
        pltpu.VMEM((128, _DW), jnp.float32),
        pltpu.VMEM((128, _DW), jnp.float32),
        pltpu.VMEM_SHARED((N_PAD, _DW), jnp.float32),
    ],
)
def _deg_kernel(ei_hbm, out_hbm, didx, ones_v, zeros_v, acc):
    c = lax.axis_index("c")
    s = lax.axis_index("s")
    wid = s * NC + c
    base = _row_base(wid)

    def fill_body(i, _):
        ones_v[i, :] = jnp.ones((16,), jnp.float32)
        zeros_v[i, :] = jnp.zeros((16,), jnp.float32)
        return _

    lax.fori_loop(0, 128, fill_body, None)

    def zcopy_body(r, _):
        pltpu.sync_copy(zeros_v, acc.at[pl.ds(s * STRIPE + r * 128, 128)])
        return _

    lax.fori_loop(0, STRIPE // 128, zcopy_body, None)
    plsc.subcore_barrier()
    pltpu.sync_copy(ei_hbm.at[1, pl.ds(base, ROWS_PT)],
                    didx.at[pl.ds(0, ROWS_PT)])

    @pl.when(wid < N_XTRA)
    def _():
        pltpu.sync_copy(ei_hbm.at[1, pl.ds(base + ROWS_PT, 1)],
                        didx.at[pl.ds(ROWS_PT, 1)])
        pltpu.sync_copy(ones_v, acc.at[didx.at[ROWS_PT]], add=True)

    def body(b, _):
        pltpu.sync_copy(ones_v, acc.at[didx.at[b]], add=True)
        return _

    lax.fori_loop(0, ROWS_PT, body, None)
    plsc.subcore_barrier()
    pltpu.sync_copy(acc.at[pl.ds(s * STRIPE, STRIPE)],
                    out_hbm.at[c, pl.ds(s * STRIPE, STRIPE)])


_NBUF = 2
_F = 64
_STEADY = ROWS_PT - _NBUF


def _make_agg64(col_off):
    @functools.partial(
        pl.kernel,
        out_type=jax.ShapeDtypeStruct((NC, N_PAD, _F), jnp.float32),
        mesh=_sc_mesh(),
        compiler_params=pltpu.CompilerParams(use_tc_tiling_on_sc=False),
        scratch_types=(
            [pltpu.VMEM((ROWS_PT + 1, 128), jnp.int32)] * 2
            + [pltpu.VMEM((128, _F), jnp.float32)] * _NBUF
            + [pltpu.VMEM_SHARED((N_PAD, _F), jnp.float32)]
            + [pltpu.VMEM_SHARED((N_PAD, _F), jnp.float32)]
            + [pltpu.SemaphoreType.DMA] * (2 * _NBUF + 2)
        ),
    )
    def _agg64(ei_hbm, p_hbm, out_hbm, sidx, didx, *scr):
        rows = scr[:_NBUF]
        p_sp = scr[_NBUF]
        acc = scr[_NBUF + 1]
        gsem = scr[_NBUF + 2:2 * _NBUF + 2]
        ssem = scr[2 * _NBUF + 2:3 * _NBUF + 2]
        isem = scr[3 * _NBUF + 2]
        psem = scr[3 * _NBUF + 3]
        c = lax.axis_index("c")
        s = lax.axis_index("s")
        wid = s * NC + c
        base = _row_base(wid)

        pltpu.async_copy(ei_hbm.at[0, pl.ds(base, ROWS_PT)],
                         sidx.at[pl.ds(0, ROWS_PT)], isem)
        pltpu.async_copy(ei_hbm.at[1, pl.ds(base, ROWS_PT)],
                         didx.at[pl.ds(0, ROWS_PT)], isem)
        pltpu.async_copy(
            p_hbm.at[pl.ds(s * STRIPE, STRIPE), pl.ds(col_off, _F)],
            p_sp.at[pl.ds(s * STRIPE, STRIPE)], psem)

        nvec = _F // 16

        def zero_body(i, _):
            rows[0][i // nvec, pl.ds((i % nvec) * 16, 16)] = jnp.zeros(
                (16,), jnp.float32)
            return _

        lax.fori_loop(0, 128 * nvec, zero_body, None)

        def zcopy_body(r, _):
            pltpu.sync_copy(rows[0], acc.at[pl.ds(s * STRIPE + r * 128, 128)])
            return _

        lax.fori_loop(0, STRIPE // 128, zcopy_body, None)
        pltpu.make_async_copy(
            ei_hbm.at[0, pl.ds(base, ROWS_PT)],
            sidx.at[pl.ds(0, ROWS_PT)], isem).wait()
        pltpu.make_async_copy(
            ei_hbm.at[1, pl.ds(base, ROWS_PT)],
            didx.at[pl.ds(0, ROWS_PT)], isem).wait()

        @pl.when(wid < N_XTRA)
        def _():
            pltpu.sync_copy(ei_hbm.at[0, pl.ds(base + ROWS_PT, 1)],
                            sidx.at[pl.ds(ROWS_PT, 1)])
            pltpu.sync_copy(ei_hbm.at[1, pl.ds(base + ROWS_PT, 1)],
                            didx.at[pl.ds(ROWS_PT, 1)])

        pltpu.make_async_copy(
            p_hbm.at[pl.ds(s * STRIPE, STRIPE), pl.ds(col_off, _F)],
            p_sp.at[pl.ds(s * STRIPE, STRIPE)], psem).wait()
        plsc.subcore_barrier()

        @pl.when(wid < N_XTRA)
        def _():
            pltpu.async_copy(p_sp.at[sidx.at[ROWS_PT]], rows[0], gsem[0])
            pltpu.make_async_copy(
                p_sp.at[sidx.at[ROWS_PT]], rows[0], gsem[0]).wait()
            pltpu.sync_copy(rows[0], acc.at[didx.at[ROWS_PT]], add=True)

        for j in range(_NBUF):
            pltpu.async_copy(p_sp.at[sidx.at[j]], rows[j], gsem[j])

        def steady(g, _):
            for j in range(_NBUF):
                b = g * _NBUF + j
                pltpu.make_async_copy(
                    p_sp.at[sidx.at[b]], rows[j], gsem[j]).wait()
                pltpu.sync_copy(rows[j], acc.at[didx.at[b]], add=True)
                pltpu.async_copy(p_sp.at[sidx.at[b + _NBUF]], rows[j],
                                 gsem[j])
            return _

        lax.fori_loop(0, _STEADY // _NBUF, steady, None)
        for j in range(_NBUF):
            b = _STEADY + j
            pltpu.make_async_copy(
                p_sp.at[sidx.at[b]], rows[j], gsem[j]).wait()
            pltpu.sync_copy(rows[j], acc.at[didx.at[b]], add=True)
        plsc.subcore_barrier()
        pltpu.sync_copy(acc.at[pl.ds(s * STRIPE, STRIPE)],
                        out_hbm.at[c, pl.ds(s * STRIPE, STRIPE)])

    return _agg64


_agg64_lo = _make_agg64(0)
_agg64_hi = _make_agg64(64)


_BLK = 1024


def _s_from_deg(deg_ref):
    d = deg_ref[0] + deg_ref[1]
    return lax.rsqrt(
        1.0 + jnp.sum(d, axis=1, keepdims=True) * (1.0 / 16.0))


def _scale_body(degT_ref, x_ref, p_ref):
    s = _s_from_deg(degT_ref)
    p_ref[...] = s * x_ref[...]


def _mid_body(agga_ref, aggb_ref, degT_ref, x_ref, w1a_ref, w1b_ref, b1_ref,
              w2_ref, zp2_ref):
    s = _s_from_deg(degT_ref)
    s2 = s * s
    x = x_ref[...]
    qa = s * (agga_ref[0] + agga_ref[1]) + s2 * x[:, :64]
    qb = s * (aggb_ref[0] + aggb_ref[1]) + s2 * x[:, 64:]
    h = jnp.dot(qa, w1a_ref[...], preferred_element_type=jnp.float32)
    h += jnp.dot(qb, w1b_ref[...], preferred_element_type=jnp.float32)
    h = jnp.maximum(h + b1_ref[...], 0.0)
    z = jnp.dot(h, w2_ref[...], preferred_element_type=jnp.float32)
    zp2_ref[...] = jnp.concatenate([z, s * z], axis=1)


def _final_body(agg_ref, degT_ref, zp2_ref, b2_ref, out_ref):
    s = _s_from_deg(degT_ref)
    z = zp2_ref[:, :64]
    out_ref[...] = s * (agg_ref[0] + agg_ref[1]) + (s * s) * z + b2_ref[...]


def _row_spec(f, blk=_BLK):
    return pl.BlockSpec((blk, f), lambda i: (i, 0))


def _agg_spec(f, blk=_BLK):
    return pl.BlockSpec((NC, blk, f), lambda i: (0, i, 0))


def _full_spec(a, b):
    return pl.BlockSpec((a, b), lambda i: (0, 0))


_GRID = (N_PAD // _BLK,)


def _tc_scale(deg_parts, x_pad):
    return pl.pallas_call(
        _scale_body,
        grid=_GRID,
        in_specs=[_agg_spec(16), _row_spec(128)],
        out_specs=_row_spec(128),
        out_shape=jax.ShapeDtypeStruct((N_PAD, 128), jnp.float32),
    )(deg_parts, x_pad)


def _tc_mid(agg1a, agg1b, deg_parts, x_pad, W1a, W1b, b1, W2):
    return pl.pallas_call(
        _mid_body,
        grid=_GRID,
        in_specs=[_agg_spec(64), _agg_spec(64), _agg_spec(16), _row_spec(128),
                  _full_spec(64, 256), _full_spec(64, 256), _full_spec(1, 256),
                  _full_spec(256, 64)],
        out_specs=_row_spec(128),
        out_shape=jax.ShapeDtypeStruct((N_PAD, 128), jnp.float32),
    )(agg1a, agg1b, deg_parts, x_pad, W1a, W1b, b1, W2)


_FBLK = 1000


def _tc_final(agg2, deg_parts, zp2, b2):
    return pl.pallas_call(
        _final_body,
        grid=(N_NODES_ // _FBLK,),
        in_specs=[_agg_spec(64, _FBLK), _agg_spec(16, _FBLK),
                  _row_spec(128, _FBLK), _full_spec(1, 64)],
        out_specs=_row_spec(64, _FBLK),
        out_shape=jax.ShapeDtypeStruct((N_NODES_, 64), jnp.float32),
    )(agg2, deg_parts, zp2, b2)


def kernel(x, edge_index, W1, b1, W2, b2):
    ei3 = edge_index.astype(jnp.int32).reshape(2, E_ROWS, 128)
    x_pad = jnp.pad(x, ((0, N_PAD - N_NODES_), (0, 0)))

    deg_parts = _deg_kernel(ei3)

    p1 = _tc_scale(deg_parts, x_pad)
    agg1a = _agg64_lo(ei3, p1)
    agg1b = _agg64_hi(ei3, p1)
    zp2 = _tc_mid(agg1a, agg1b, deg_parts, x_pad, W1[:64], W1[64:],
                  b1.reshape(1, 256), W2)
    agg2 = _agg64_hi(ei3, zp2)
    return _tc_final(agg2, deg_parts, zp2, b2.reshape(1, 64))


# speedup vs baseline: 1.1282x; 1.0023x over previous
import functools

import jax
import jax.numpy as jnp
from jax import lax
from jax.experimental import pallas as pl
from jax.experimental.pallas import tpu as pltpu
from jax.experimental.pallas import tpu_sc as plsc

N_NODES_ = 10000
N_PAD = 10240
E_EDGES = 320000
E_ROWS = E_EDGES // 128
NC, NS = 2, 16
NW = NC * NS
ROWS_PT = E_ROWS // NW
N_XTRA = E_ROWS - NW * ROWS_PT
STRIPE = N_PAD // NS


def _row_base(wid):
    return jnp.where(wid < N_XTRA, wid * (ROWS_PT + 1),
                     wid * ROWS_PT + N_XTRA)

_sc_mesh = functools.partial(
    plsc.VectorSubcoreMesh, core_axis_name="c", subcore_axis_name="s")


_DW = 16


@functools.partial(
    pl.kernel,
    out_type=jax.ShapeDtypeStruct((NC, N_PAD, _DW), jnp.float32),
    mesh=_sc_mesh(),
    compiler_params=pltpu.CompilerParams(use_tc_tiling_on_sc=False),
    scratch_types=[
        pltpu.VMEM((ROWS_PT + 1, 128), jnp.int32),
        pltpu.VMEM((128, _DW), jnp.float32),
        pltpu.VMEM((128, _DW), jnp.float32),
        pltpu.VMEM_SHARED((N_PAD, _DW), jnp.float32),
    ],
)
def _deg_kernel(ei_hbm, out_hbm, didx, ones_v, zeros_v, acc):
    c = lax.axis_index("c")
    s = lax.axis_index("s")
    wid = s * NC + c
    base = _row_base(wid)

    def fill_body(i, _):
        ones_v[i, :] = jnp.ones((16,), jnp.float32)
        zeros_v[i, :] = jnp.zeros((16,), jnp.float32)
        return _

    lax.fori_loop(0, 128, fill_body, None)

    def zcopy_body(r, _):
        pltpu.sync_copy(zeros_v, acc.at[pl.ds(s * STRIPE + r * 128, 128)])
        return _

    lax.fori_loop(0, STRIPE // 128, zcopy_body, None)
    plsc.subcore_barrier()
    pltpu.sync_copy(ei_hbm.at[1, pl.ds(base, ROWS_PT)],
                    didx.at[pl.ds(0, ROWS_PT)])

    @pl.when(wid < N_XTRA)
    def _():
        pltpu.sync_copy(ei_hbm.at[1, pl.ds(base + ROWS_PT, 1)],
                        didx.at[pl.ds(ROWS_PT, 1)])
        pltpu.sync_copy(ones_v, acc.at[didx.at[ROWS_PT]], add=True)

    def body(b, _):
        pltpu.sync_copy(ones_v, acc.at[didx.at[b]], add=True)
        return _

    lax.fori_loop(0, ROWS_PT, body, None)
    plsc.subcore_barrier()
    pltpu.sync_copy(acc.at[pl.ds(s * STRIPE, STRIPE)],
                    out_hbm.at[c, pl.ds(s * STRIPE, STRIPE)])


_NBUF = 2
_F = 64
_STEADY = ROWS_PT - _NBUF


def _make_agg64(col_off):
    @functools.partial(
        pl.kernel,
        out_type=jax.ShapeDtypeStruct((NC, N_PAD, _F), jnp.float32),
        mesh=_sc_mesh(),
        compiler_params=pltpu.CompilerParams(use_tc_tiling_on_sc=False),
        scratch_types=(
            [pltpu.VMEM((ROWS_PT + 1, 128), jnp.int32)] * 2
            + [pltpu.VMEM((128, _F), jnp.float32)] * _NBUF
            + [pltpu.VMEM_SHARED((N_PAD, _F), jnp.float32)]
            + [pltpu.VMEM_SHARED((N_PAD, _F), jnp.float32)]
            + [pltpu.SemaphoreType.DMA] * (2 * _NBUF + 2)
        ),
    )
    def _agg64(ei_hbm, p_hbm, out_hbm, sidx, didx, *scr):
        rows = scr[:_NBUF]
        p_sp = scr[_NBUF]
        acc = scr[_NBUF + 1]
        gsem = scr[_NBUF + 2:2 * _NBUF + 2]
        ssem = scr[2 * _NBUF + 2:3 * _NBUF + 2]
        isem = scr[3 * _NBUF + 2]
        psem = scr[3 * _NBUF + 3]
        c = lax.axis_index("c")
        s = lax.axis_index("s")
        wid = s * NC + c
        base = _row_base(wid)

        pltpu.async_copy(ei_hbm.at[0, pl.ds(base, ROWS_PT)],
                         sidx.at[pl.ds(0, ROWS_PT)], isem)
        pltpu.async_copy(ei_hbm.at[1, pl.ds(base, ROWS_PT)],
                         didx.at[pl.ds(0, ROWS_PT)], isem)
        pltpu.async_copy(
            p_hbm.at[pl.ds(s * STRIPE, STRIPE), pl.ds(col_off, _F)],
            p_sp.at[pl.ds(s * STRIPE, STRIPE)], psem)

        nvec = _F // 16

        def zero_body(i, _):
            rows[0][i // nvec, pl.ds((i % nvec) * 16, 16)] = jnp.zeros(
                (16,), jnp.float32)
            return _

        lax.fori_loop(0, 128 * nvec, zero_body, None)

        def zcopy_body(r, _):
            pltpu.sync_copy(rows[0], acc.at[pl.ds(s * STRIPE + r * 128, 128)])
            return _

        lax.fori_loop(0, STRIPE // 128, zcopy_body, None)
        pltpu.make_async_copy(
            ei_hbm.at[0, pl.ds(base, ROWS_PT)],
            sidx.at[pl.ds(0, ROWS_PT)], isem).wait()
        pltpu.make_async_copy(
            ei_hbm.at[1, pl.ds(base, ROWS_PT)],
            didx.at[pl.ds(0, ROWS_PT)], isem).wait()

        @pl.when(wid < N_XTRA)
        def _():
            pltpu.sync_copy(ei_hbm.at[0, pl.ds(base + ROWS_PT, 1)],
                            sidx.at[pl.ds(ROWS_PT, 1)])
            pltpu.sync_copy(ei_hbm.at[1, pl.ds(base + ROWS_PT, 1)],
                            didx.at[pl.ds(ROWS_PT, 1)])

        pltpu.make_async_copy(
            p_hbm.at[pl.ds(s * STRIPE, STRIPE), pl.ds(col_off, _F)],
            p_sp.at[pl.ds(s * STRIPE, STRIPE)], psem).wait()
        plsc.subcore_barrier()

        @pl.when(wid < N_XTRA)
        def _():
            pltpu.async_copy(p_sp.at[sidx.at[ROWS_PT]], rows[0], gsem[0])
            pltpu.make_async_copy(
                p_sp.at[sidx.at[ROWS_PT]], rows[0], gsem[0]).wait()
            pltpu.sync_copy(rows[0], acc.at[didx.at[ROWS_PT]], add=True)

        for j in range(_NBUF):
            pltpu.async_copy(p_sp.at[sidx.at[j]], rows[j], gsem[j])

        def steady(g, _):
            for j in range(_NBUF):
                b = g * _NBUF + j
                pltpu.make_async_copy(
                    p_sp.at[sidx.at[b]], rows[j], gsem[j]).wait()
                pltpu.sync_copy(rows[j], acc.at[didx.at[b]], add=True)
                pltpu.async_copy(p_sp.at[sidx.at[b + _NBUF]], rows[j],
                                 gsem[j])
            return _

        lax.fori_loop(0, _STEADY // _NBUF, steady, None)
        for j in range(_NBUF):
            b = _STEADY + j
            pltpu.make_async_copy(
                p_sp.at[sidx.at[b]], rows[j], gsem[j]).wait()
            pltpu.sync_copy(rows[j], acc.at[didx.at[b]], add=True)
        plsc.subcore_barrier()
        pltpu.sync_copy(acc.at[pl.ds(s * STRIPE, STRIPE)],
                        out_hbm.at[c, pl.ds(s * STRIPE, STRIPE)])

    return _agg64


_agg64_lo = _make_agg64(0)
_agg64_hi = _make_agg64(64)


_BLK = 1024


def _s_from_deg(deg_ref):
    d = deg_ref[0] + deg_ref[1]
    return lax.rsqrt(
        1.0 + jnp.sum(d, axis=1, keepdims=True) * (1.0 / 16.0))


def _scale_body(degT_ref, x_ref, p_ref):
    s = _s_from_deg(degT_ref)
    p_ref[...] = s * x_ref[...]


def _mid_body(agga_ref, aggb_ref, degT_ref, x_ref, w1a_ref, w1b_ref, b1_ref,
              w2_ref, zp2_ref):
    s = _s_from_deg(degT_ref)
    s2 = s * s
    x = x_ref[...]
    qa = s * (agga_ref[0] + agga_ref[1]) + s2 * x[:, :64]
    qb = s * (aggb_ref[0] + aggb_ref[1]) + s2 * x[:, 64:]
    h = jnp.dot(qa, w1a_ref[...], preferred_element_type=jnp.float32)
    h += jnp.dot(qb, w1b_ref[...], preferred_element_type=jnp.float32)
    h = jnp.maximum(h + b1_ref[...], 0.0)
    z = jnp.dot(h, w2_ref[...], preferred_element_type=jnp.float32)
    zp2_ref[...] = jnp.concatenate([z, s * z], axis=1)


def _final_body(agg_ref, degT_ref, zp2_ref, b2_ref, out_ref):
    s = _s_from_deg(degT_ref)
    z = zp2_ref[:, :64]
    out_ref[...] = s * (agg_ref[0] + agg_ref[1]) + (s * s) * z + b2_ref[...]


def _row_spec(f, blk=_BLK):
    return pl.BlockSpec((blk, f), lambda i: (i, 0))


def _agg_spec(f, blk=_BLK):
    return pl.BlockSpec((NC, blk, f), lambda i: (0, i, 0))


def _full_spec(a, b):
    return pl.BlockSpec((a, b), lambda i: (0, 0))


_GRID = (N_PAD // _BLK,)


def _tc_scale(deg_parts, x_pad):
    return pl.pallas_call(
        _scale_body,
        grid=_GRID,
        in_specs=[_agg_spec(16), _row_spec(128)],
        out_specs=_row_spec(128),
        out_shape=jax.ShapeDtypeStruct((N_PAD, 128), jnp.float32),
    )(deg_parts, x_pad)


def _tc_mid(agg1a, agg1b, deg_parts, x_pad, W1a, W1b, b1, W2):
    return pl.pallas_call(
        _mid_body,
        grid=_GRID,
        in_specs=[_agg_spec(64), _agg_spec(64), _agg_spec(16), _row_spec(128),
                  _full_spec(64, 256), _full_spec(64, 256), _full_spec(1, 256),
                  _full_spec(256, 64)],
        out_specs=_row_spec(128),
        out_shape=jax.ShapeDtypeStruct((N_PAD, 128), jnp.float32),
    )(agg1a, agg1b, deg_parts, x_pad, W1a, W1b, b1, W2)


_FBLK = 1000


def _tc_final(agg2, deg_parts, zp2, b2):
    return pl.pallas_call(
        _final_body,
        grid=(N_NODES_ // _FBLK,),
        in_specs=[_agg_spec(64, _FBLK), _agg_spec(16, _FBLK),
                  _row_spec(128, _FBLK), _full_spec(1, 64)],
        out_specs=_row_spec(64, _FBLK),
        out_shape=jax.ShapeDtypeStruct((N_NODES_, 64), jnp.float32),
    )(agg2, deg_parts, zp2, b2)


def kernel(x, edge_index, W1, b1, W2, b2):
    ei3 = edge_index.astype(jnp.int32).reshape(2, E_ROWS, 128)

    deg_parts = _deg_kernel(ei3)

    p1 = _tc_scale(deg_parts, x)
    agg1a = _agg64_lo(ei3, p1)
    agg1b = _agg64_hi(ei3, p1)
    zp2 = _tc_mid(agg1a, agg1b, deg_parts, x, W1[:64], W1[64:],
                  b1.reshape(1, 256), W2)
    agg2 = _agg64_hi(ei3, zp2)
    return _tc_final(agg2, deg_parts, zp2, b2.reshape(1, 64))
